# Initial kernel scaffold; baseline (speedup 1.0000x reference)
import jax
import jax.numpy as jnp
from jax.experimental import pallas as pl


def kernel(token_ids, table):
    raise NotImplementedError("write your pallas kernel here")


# speedup vs baseline: 1.5734x; 1.5734x over previous
import functools

import jax
import jax.numpy as jnp
from jax import lax
from jax.experimental import pallas as pl
from jax.experimental.pallas import tpu as pltpu
from jax.experimental.pallas import tpu_sc as plsc

_LANES = 16


def _make_sc_kernel(B, L, D, V):
    info = plsc.get_sparse_core_info()
    NC, NS = info.num_cores, info.num_subcores
    NW = NC * NS
    assert B % NW == 0
    b_per_w = B // NW
    n_d = D // _LANES
    half = L // 2
    OUT = 3 * D

    mesh = plsc.VectorSubcoreMesh(core_axis_name="c", subcore_axis_name="s")

    @functools.partial(
        pl.kernel,
        mesh=mesh,
        compiler_params=pltpu.CompilerParams(use_tc_tiling_on_sc=False),
        out_type=jax.ShapeDtypeStruct((B, OUT), jnp.float32),
        scratch_types=[
            pltpu.VMEM((b_per_w, 2, half), jnp.int32),
            pltpu.VMEM((L, D), jnp.float32),
            pltpu.VMEM((b_per_w, OUT), jnp.float32),
            pltpu.SemaphoreType.DMA,
        ],
    )
    def k(tok_hbm, table_hbm, out_hbm, idx_v, rows_v, out_v, sem):
        wid = lax.axis_index("s") * NC + lax.axis_index("c")
        base = wid * b_per_w

        pltpu.sync_copy(tok_hbm.at[pl.ds(base, b_per_w)], idx_v)

        inv1 = 1.0 / L
        inv2 = 1.0 / (L - 1)
        inv3 = 1.0 / (L - 2)

        def row_body(i, _):
            cp0 = pltpu.async_copy(
                table_hbm.at[idx_v.at[i, 0]], rows_v.at[pl.ds(0, half)], sem)
            cp1 = pltpu.async_copy(
                table_hbm.at[idx_v.at[i, 1]], rows_v.at[pl.ds(half, half)], sem)
            cp0.wait()
            cp1.wait()

            def step(l, carry):
                new = []
                for c in range(n_d):
                    e_prev, pair_prev, a1, a2, a3 = carry[5 * c:5 * c + 5]
                    e = rows_v[l, pl.ds(c * _LANES, _LANES)]
                    pair = e_prev * e
                    trip = pair_prev * e
                    new.extend((e, pair, a1 + e, a2 + pair, a3 + trip))
                return tuple(new)

            zeros = jnp.zeros((_LANES,), jnp.float32)
            carry = tuple(zeros for _ in range(5 * n_d))
            carry = lax.fori_loop(0, L, step, carry)
            for c in range(n_d):
                _, _, a1, a2, a3 = carry[5 * c:5 * c + 5]
                out_v[i, pl.ds(c * _LANES, _LANES)] = a1 * inv1
                out_v[i, pl.ds(D + c * _LANES, _LANES)] = a2 * inv2
                out_v[i, pl.ds(2 * D + c * _LANES, _LANES)] = a3 * inv3
            return None

        lax.fori_loop(0, b_per_w, row_body, None)

        pltpu.sync_copy(out_v, out_hbm.at[pl.ds(base, b_per_w)])

    return k


def kernel(token_ids, table):
    B, L = token_ids.shape
    V, D = table.shape
    tok3 = token_ids.reshape(B, 2, L // 2).astype(jnp.int32)
    k = _make_sc_kernel(B, L, D, V)
    return k(tok3, table)


# speedup vs baseline: 1.8472x; 1.1740x over previous
import functools

import jax
import jax.numpy as jnp
from jax import lax
from jax.experimental import pallas as pl
from jax.experimental.pallas import tpu as pltpu
from jax.experimental.pallas import tpu_sc as plsc

_LANES = 16


def _make_sc_kernel(B, L, D, V):
    info = plsc.get_sparse_core_info()
    NC, NS = info.num_cores, info.num_subcores
    NW = NC * NS
    assert B % NW == 0
    b_per_w = B // NW
    n_d = D // _LANES
    half = L // 2
    OUT = 3 * D

    mesh = plsc.VectorSubcoreMesh(core_axis_name="c", subcore_axis_name="s")

    @functools.partial(
        pl.kernel,
        mesh=mesh,
        compiler_params=pltpu.CompilerParams(use_tc_tiling_on_sc=False),
        out_type=jax.ShapeDtypeStruct((B, OUT), jnp.float32),
        scratch_types=[
            pltpu.VMEM((b_per_w, 2, half), jnp.int32),
            pltpu.VMEM((2, L, D), jnp.float32),
            pltpu.VMEM((b_per_w, OUT), jnp.float32),
            pltpu.SemaphoreType.DMA,
            pltpu.SemaphoreType.DMA,
        ],
    )
    def k(tok_hbm, table_hbm, out_hbm, idx_v, rows_v, out_v, sem0, sem1):
        wid = lax.axis_index("s") * NC + lax.axis_index("c")
        base = wid * b_per_w

        pltpu.sync_copy(tok_hbm.at[pl.ds(base, b_per_w)], idx_v)

        inv1 = 1.0 / L
        inv2 = 1.0 / (L - 1)
        inv3 = 1.0 / (L - 2)

        def issue(i, b, sem):
            pltpu.async_copy(
                table_hbm.at[idx_v.at[i, 0]], rows_v.at[b, pl.ds(0, half)], sem)
            pltpu.async_copy(
                table_hbm.at[idx_v.at[i, 1]], rows_v.at[b, pl.ds(half, half)], sem)

        def drain(b, sem):
            for s in range(2):
                pltpu.make_async_copy(
                    table_hbm.at[idx_v.at[0, 0]],
                    rows_v.at[b, pl.ds(s * half, half)], sem).wait()

        def compute(i, b):
            def step(l, carry):
                new = []
                for c in range(n_d):
                    e_prev, pair_prev, a1, a2, a3 = carry[5 * c:5 * c + 5]
                    e = rows_v[b, l, pl.ds(c * _LANES, _LANES)]
                    pair = e_prev * e
                    trip = pair_prev * e
                    new.extend((e, pair, a1 + e, a2 + pair, a3 + trip))
                return tuple(new)

            zeros = jnp.zeros((_LANES,), jnp.float32)
            carry = tuple(zeros for _ in range(5 * n_d))
            carry = lax.fori_loop(0, L, step, carry)
            for c in range(n_d):
                _, _, a1, a2, a3 = carry[5 * c:5 * c + 5]
                out_v[i, pl.ds(c * _LANES, _LANES)] = a1 * inv1
                out_v[i, pl.ds(D + c * _LANES, _LANES)] = a2 * inv2
                out_v[i, pl.ds(2 * D + c * _LANES, _LANES)] = a3 * inv3

        issue(0, 0, sem0)

        def pair_body(j, _):
            i0 = 2 * j
            issue(i0 + 1, 1, sem1)
            drain(0, sem0)
            compute(i0, 0)

            @pl.when(i0 + 2 < b_per_w)
            def _():
                issue(i0 + 2, 0, sem0)

            drain(1, sem1)
            compute(i0 + 1, 1)
            return None

        lax.fori_loop(0, b_per_w // 2, pair_body, None)

        pltpu.sync_copy(out_v, out_hbm.at[pl.ds(base, b_per_w)])

    return k


def kernel(token_ids, table):
    B, L = token_ids.shape
    V, D = table.shape
    tok3 = token_ids.reshape(B, 2, L // 2).astype(jnp.int32)
    k = _make_sc_kernel(B, L, D, V)
    return k(tok3, table)


# speedup vs baseline: 1.8801x; 1.0178x over previous
import functools

import jax
import jax.numpy as jnp
from jax import lax
from jax.experimental import pallas as pl
from jax.experimental.pallas import tpu as pltpu
from jax.experimental.pallas import tpu_sc as plsc

_LANES = 16


def _make_sc_kernel(B, L, D, V):
    info = plsc.get_sparse_core_info()
    NC, NS = info.num_cores, info.num_subcores
    NW = NC * NS
    assert B % NW == 0
    b_per_w = B // NW
    n_d = D // _LANES
    half = L // 2
    OUT = 3 * D

    mesh = plsc.VectorSubcoreMesh(core_axis_name="c", subcore_axis_name="s")

    @functools.partial(
        pl.kernel,
        mesh=mesh,
        compiler_params=pltpu.CompilerParams(use_tc_tiling_on_sc=False),
        out_type=jax.ShapeDtypeStruct((B, OUT), jnp.float32),
        scratch_types=[
            pltpu.VMEM((b_per_w, 2, half), jnp.int32),
            pltpu.VMEM((2, L, D), jnp.float32),
            pltpu.VMEM((b_per_w, OUT), jnp.float32),
            pltpu.SemaphoreType.DMA,
            pltpu.SemaphoreType.DMA,
        ],
    )
    def k(tok_hbm, table_hbm, out_hbm, idx_v, rows_v, out_v, sem0, sem1):
        wid = lax.axis_index("s") * NC + lax.axis_index("c")
        base = wid * b_per_w

        pltpu.sync_copy(tok_hbm.at[pl.ds(base, b_per_w)], idx_v)

        inv1 = 1.0 / L
        inv2 = 1.0 / (L - 1)
        inv3 = 1.0 / (L - 2)

        def issue(i, b, sem):
            pltpu.async_copy(
                table_hbm.at[idx_v.at[i, 0]], rows_v.at[b, pl.ds(0, half)], sem)
            pltpu.async_copy(
                table_hbm.at[idx_v.at[i, 1]], rows_v.at[b, pl.ds(half, half)], sem)

        def drain(b, sem):
            for s in range(2):
                pltpu.make_async_copy(
                    table_hbm.at[idx_v.at[0, 0]],
                    rows_v.at[b, pl.ds(s * half, half)], sem).wait()

        def compute(i, b):
            def step(l, carry):
                new = []
                for c in range(n_d):
                    e_prev, pair_prev, a1, a2, a3 = carry[5 * c:5 * c + 5]
                    e = rows_v[b, l, pl.ds(c * _LANES, _LANES)]
                    pair = e_prev * e
                    trip = pair_prev * e
                    new.extend((e, pair, a1 + e, a2 + pair, a3 + trip))
                return tuple(new)

            zeros = jnp.zeros((_LANES,), jnp.float32)
            carry = tuple(zeros for _ in range(5 * n_d))
            carry = lax.fori_loop(0, L, step, carry, unroll=8)
            for c in range(n_d):
                _, _, a1, a2, a3 = carry[5 * c:5 * c + 5]
                out_v[i, pl.ds(c * _LANES, _LANES)] = a1 * inv1
                out_v[i, pl.ds(D + c * _LANES, _LANES)] = a2 * inv2
                out_v[i, pl.ds(2 * D + c * _LANES, _LANES)] = a3 * inv3

        issue(0, 0, sem0)

        def pair_body(j, _):
            i0 = 2 * j
            issue(i0 + 1, 1, sem1)
            drain(0, sem0)
            compute(i0, 0)

            @pl.when(i0 + 2 < b_per_w)
            def _():
                issue(i0 + 2, 0, sem0)

            drain(1, sem1)
            compute(i0 + 1, 1)
            return None

        lax.fori_loop(0, b_per_w // 2, pair_body, None)

        pltpu.sync_copy(out_v, out_hbm.at[pl.ds(base, b_per_w)])

    return k


def kernel(token_ids, table):
    B, L = token_ids.shape
    V, D = table.shape
    tok3 = token_ids.reshape(B, 2, L // 2).astype(jnp.int32)
    k = _make_sc_kernel(B, L, D, V)
    return k(tok3, table)
